# baseline (speedup 1.0000x reference)
import functools

import jax
import jax.numpy as jnp
from jax import lax
from jax.experimental import pallas as pl
from jax.experimental.pallas import tpu as pltpu
from jax.experimental.pallas import tpu_sc as plsc

_B = 1024
_F = 1000
_FP = 1008
_K = 32

_NC = 2
_NS = 16
_NW = _NC * _NS
_L = 16

_BH = _B // 2
_RH = _BH // _NW

_BBLK = 256


def _make_sc_half_body(half):
    def _sc_hist_body(x_hbm, counts_hbm, x_v, hist_v, sem0):
        wid = lax.axis_index("s") * _NC + lax.axis_index("c")
        base = half * _BH + wid * _RH
        cp = pltpu.async_copy(x_hbm.at[pl.ds(base, _RH)], x_v, sem0)

        zeros = jnp.zeros((_L,), jnp.float32)

        @plsc.parallel_loop(0, _RH, 1, unroll=2)
        def _zero(r):
            for c in range(_FP // _L):
                hist_v[r, pl.ds(c * _L, _L)] = zeros

        ones = jnp.ones((_L,), jnp.float32)
        lane = lax.iota(jnp.int32, _L)

        cp.wait()

        @plsc.parallel_loop(0, _F, 1, unroll=8)
        def _feat(f):
            fv = jnp.full((_L,), f, dtype=jnp.int32)
            ids = plsc.load_gather(x_v, [lane, fv])
            plsc.addupdate_scatter(hist_v, [lane, ids], ones)

        pltpu.sync_copy(hist_v, counts_hbm.at[pl.ds(wid * _RH, _RH)])

    return _sc_hist_body


def _sc_counts_half(x, half):
    mesh = plsc.VectorSubcoreMesh(core_axis_name="c", subcore_axis_name="s")
    return pl.kernel(
        _make_sc_half_body(half),
        out_type=jax.ShapeDtypeStruct((_BH, _FP), jnp.float32),
        mesh=mesh,
        compiler_params=pltpu.CompilerParams(
            use_tc_tiling_on_sc=False, needs_layout_passes=False),
        scratch_types=[
            pltpu.VMEM((_RH, _F), jnp.int32),
            pltpu.VMEM((_RH, _FP), jnp.float32),
            pltpu.SemaphoreType.DMA,
        ],
    )(x)


def _tc_body(counts_ref, emb_ref, w1_ref, b1_ref, w2_ref, b2_ref, w3_ref,
             b3_ref, out_ref, embp_v):
    @pl.when(pl.program_id(0) == 0)
    def _prep():
        emb = emb_ref[...]
        ee = jnp.concatenate([emb, emb * emb], axis=1)
        pad = jnp.zeros((_FP - _F, 2 * _K), jnp.float32)
        embp_v[...] = jnp.concatenate([ee, pad], axis=0)

    sb = jnp.dot(counts_ref[...], embp_v[...])
    s = sb[:, :_K]
    ss = sb[:, _K:]
    bi = 0.5 * (s * s - ss)
    h = jnp.maximum(jnp.dot(bi, w1_ref[...]) + b1_ref[...], 0.0)
    h = jnp.maximum(jnp.dot(h, w2_ref[...]) + b2_ref[...], 0.0)
    out_ref[...] = jnp.dot(h, w3_ref[...]) + b3_ref[...]


def _tc_mlp(counts, emb, W1, b1, W2, b2, W3, b3):
    nblk = counts.shape[0] // _BBLK
    full = lambda shape: pl.BlockSpec(shape, lambda i: (0, 0))
    return pl.pallas_call(
        _tc_body,
        grid=(nblk,),
        in_specs=[
            pl.BlockSpec((_BBLK, _FP), lambda i: (i, 0)),
            full((_F, _K)),
            full(W1.shape), full(b1.shape),
            full(W2.shape), full(b2.shape),
            full(W3.shape), full(b3.shape),
        ],
        out_specs=pl.BlockSpec((_BBLK, 1), lambda i: (i, 0)),
        out_shape=jax.ShapeDtypeStruct((counts.shape[0], 1), jnp.float32),
        scratch_shapes=[pltpu.VMEM((_FP, 2 * _K), jnp.float32)],
    )(counts, emb, W1, b1, W2, b2, W3, b3)


@jax.jit
def kernel(x, emb, W1, b1, W2, b2, W3, b3):
    xi = x.astype(jnp.int32)
    b1r = b1.reshape(1, -1)
    b2r = b2.reshape(1, -1)
    b3r = b3.reshape(1, 1)
    counts_a = _sc_counts_half(xi, 0)
    counts_b = _sc_counts_half(xi, 1)
    out_a = _tc_mlp(counts_a, emb, W1, b1r, W2, b2r, W3, b3r)
    out_b = _tc_mlp(counts_b, emb, W1, b1r, W2, b2r, W3, b3r)
    return jnp.concatenate([out_a, out_b], axis=0)


# speedup vs baseline: 1.0431x; 1.0431x over previous
import functools

import jax
import jax.numpy as jnp
from jax import lax
from jax.experimental import pallas as pl
from jax.experimental.pallas import tpu as pltpu
from jax.experimental.pallas import tpu_sc as plsc

_B = 1024
_F = 1000
_FP = 1008
_K = 32

_NC = 2
_NS = 16
_NW = _NC * _NS
_ROWS = _B // _NW
_L = 16

_BBLK = 256


def _sc_hist_body(x_hbm, counts_hbm, x_v, hist_v, sem0, sem1, sem2):
    wid = lax.axis_index("s") * _NC + lax.axis_index("c")
    base = wid * _ROWS
    cp0 = pltpu.async_copy(x_hbm.at[pl.ds(base, _L)], x_v.at[pl.ds(0, _L)],
                           sem0)
    cp1 = pltpu.async_copy(x_hbm.at[pl.ds(base + _L, _L)],
                           x_v.at[pl.ds(_L, _L)], sem1)

    zeros = jnp.zeros((_L,), jnp.float32)

    @plsc.parallel_loop(0, _ROWS, 1, unroll=2)
    def _zero(r):
        for c in range(_FP // _L):
            hist_v[r, pl.ds(c * _L, _L)] = zeros

    ones = jnp.ones((_L,), jnp.float32)
    lane = lax.iota(jnp.int32, _L)

    cp0.wait()

    @plsc.parallel_loop(0, _F, 1, unroll=8)
    def _feat0(f):
        fv = jnp.full((_L,), f, dtype=jnp.int32)
        ids = plsc.load_gather(x_v, [lane, fv])
        plsc.addupdate_scatter(hist_v, [lane, ids], ones)

    out0 = pltpu.async_copy(hist_v.at[pl.ds(0, _L)],
                            counts_hbm.at[pl.ds(base, _L)], sem2)
    cp1.wait()
    rows1 = lane + _L

    @plsc.parallel_loop(0, _F, 1, unroll=8)
    def _feat1(f):
        fv = jnp.full((_L,), f, dtype=jnp.int32)
        ids = plsc.load_gather(x_v, [rows1, fv])
        plsc.addupdate_scatter(hist_v, [rows1, ids], ones)

    out0.wait()
    pltpu.sync_copy(hist_v.at[pl.ds(_L, _L)],
                    counts_hbm.at[pl.ds(base + _L, _L)])


def _sc_counts(x):
    mesh = plsc.VectorSubcoreMesh(core_axis_name="c", subcore_axis_name="s")
    return pl.kernel(
        _sc_hist_body,
        out_type=jax.ShapeDtypeStruct((_B, _FP), jnp.float32),
        mesh=mesh,
        compiler_params=pltpu.CompilerParams(
            use_tc_tiling_on_sc=False, needs_layout_passes=False),
        scratch_types=[
            pltpu.VMEM((_ROWS, _F), jnp.int32),
            pltpu.VMEM((_ROWS, _FP), jnp.float32),
            pltpu.SemaphoreType.DMA,
            pltpu.SemaphoreType.DMA,
            pltpu.SemaphoreType.DMA,
        ],
    )(x)


def _tc_body(counts_ref, emb_ref, w1_ref, b1_ref, w2_ref, b2_ref, w3_ref,
             b3_ref, out_ref, embp_v):
    @pl.when(pl.program_id(0) == 0)
    def _prep():
        emb = emb_ref[...]
        ee = jnp.concatenate([emb, emb * emb], axis=1)
        pad = jnp.zeros((_FP - _F, 2 * _K), jnp.float32)
        eef = jnp.concatenate([ee, pad], axis=0)
        e_hi = eef.astype(jnp.bfloat16)
        e_lo = (eef - e_hi.astype(jnp.float32)).astype(jnp.bfloat16)
        embp_v[...] = jnp.concatenate([e_hi, e_lo], axis=1)

    c16 = counts_ref[...].astype(jnp.bfloat16)
    sb2 = jnp.dot(c16, embp_v[...],
                  preferred_element_type=jnp.float32)
    sb = sb2[:, :2 * _K] + sb2[:, 2 * _K:]
    s = sb[:, :_K]
    ss = sb[:, _K:]
    bi = 0.5 * (s * s - ss)
    h = jnp.maximum(jnp.dot(bi, w1_ref[...]) + b1_ref[...], 0.0)
    h = jnp.maximum(jnp.dot(h, w2_ref[...]) + b2_ref[...], 0.0)
    out_ref[...] = jnp.dot(h, w3_ref[...]) + b3_ref[...]


def _tc_mlp(counts, emb, W1, b1, W2, b2, W3, b3):
    nblk = _B // _BBLK
    full = lambda shape: pl.BlockSpec(shape, lambda i: (0, 0))
    return pl.pallas_call(
        _tc_body,
        grid=(nblk,),
        in_specs=[
            pl.BlockSpec((_BBLK, _FP), lambda i: (i, 0)),
            full((_F, _K)),
            full(W1.shape), full(b1.shape),
            full(W2.shape), full(b2.shape),
            full(W3.shape), full(b3.shape),
        ],
        out_specs=pl.BlockSpec((_BBLK, 1), lambda i: (i, 0)),
        out_shape=jax.ShapeDtypeStruct((_B, 1), jnp.float32),
        scratch_shapes=[pltpu.VMEM((_FP, 4 * _K), jnp.bfloat16)],
    )(counts, emb, W1, b1, W2, b2, W3, b3)


@jax.jit
def kernel(x, emb, W1, b1, W2, b2, W3, b3):
    counts = _sc_counts(x.astype(jnp.int32))
    return _tc_mlp(counts, emb, W1,
                   b1.reshape(1, -1), W2, b2.reshape(1, -1), W3,
                   b3.reshape(1, 1))


# speedup vs baseline: 1.0741x; 1.0297x over previous
import functools

import jax
import jax.numpy as jnp
from jax import lax
from jax.experimental import pallas as pl
from jax.experimental.pallas import tpu as pltpu
from jax.experimental.pallas import tpu_sc as plsc

_B = 1024
_F = 1000
_FP = 1008
_K = 32

_NC = 2
_NS = 16
_NW = _NC * _NS
_ROWS = _B // _NW
_L = 16

_BBLK = 512


def _sc_hist_body(x_hbm, counts_hbm, x_v, hist_v, sem0, sem1, sem2):
    wid = lax.axis_index("s") * _NC + lax.axis_index("c")
    base = wid * _ROWS
    cp0 = pltpu.async_copy(x_hbm.at[pl.ds(base, _L)], x_v.at[pl.ds(0, _L)],
                           sem0)
    cp1 = pltpu.async_copy(x_hbm.at[pl.ds(base + _L, _L)],
                           x_v.at[pl.ds(_L, _L)], sem1)

    zeros = jnp.zeros((_L,), jnp.float32)

    @plsc.parallel_loop(0, _ROWS, 1, unroll=2)
    def _zero(r):
        for c in range(_FP // _L):
            hist_v[r, pl.ds(c * _L, _L)] = zeros

    ones = jnp.ones((_L,), jnp.float32)
    lane = lax.iota(jnp.int32, _L)

    cp0.wait()

    @plsc.parallel_loop(0, _F, 1, unroll=8)
    def _feat0(f):
        fv = jnp.full((_L,), f, dtype=jnp.int32)
        ids = plsc.load_gather(x_v, [lane, fv])
        plsc.addupdate_scatter(hist_v, [lane, ids], ones)

    out0 = pltpu.async_copy(hist_v.at[pl.ds(0, _L)],
                            counts_hbm.at[pl.ds(base, _L)], sem2)
    cp1.wait()
    rows1 = lane + _L

    @plsc.parallel_loop(0, _F, 1, unroll=8)
    def _feat1(f):
        fv = jnp.full((_L,), f, dtype=jnp.int32)
        ids = plsc.load_gather(x_v, [rows1, fv])
        plsc.addupdate_scatter(hist_v, [rows1, ids], ones)

    out0.wait()
    pltpu.sync_copy(hist_v.at[pl.ds(_L, _L)],
                    counts_hbm.at[pl.ds(base + _L, _L)])


def _sc_counts(x):
    mesh = plsc.VectorSubcoreMesh(core_axis_name="c", subcore_axis_name="s")
    return pl.kernel(
        _sc_hist_body,
        out_type=jax.ShapeDtypeStruct((_B, _FP), jnp.float32),
        mesh=mesh,
        compiler_params=pltpu.CompilerParams(
            use_tc_tiling_on_sc=False, needs_layout_passes=False),
        scratch_types=[
            pltpu.VMEM((_ROWS, _F), jnp.int32),
            pltpu.VMEM((_ROWS, _FP), jnp.float32),
            pltpu.SemaphoreType.DMA,
            pltpu.SemaphoreType.DMA,
            pltpu.SemaphoreType.DMA,
        ],
    )(x)


def _tc_body(counts_ref, emb_ref, w1_ref, b1_ref, w2_ref, b2_ref, w3_ref,
             b3_ref, out_ref, embp_v):
    @pl.when(pl.program_id(0) == 0)
    def _prep():
        emb = emb_ref[...]
        ee = jnp.concatenate([emb, emb * emb], axis=1)
        pad = jnp.zeros((_FP - _F, 2 * _K), jnp.float32)
        eef = jnp.concatenate([ee, pad], axis=0)
        e_hi = eef.astype(jnp.bfloat16)
        e_lo = (eef - e_hi.astype(jnp.float32)).astype(jnp.bfloat16)
        embp_v[...] = jnp.concatenate([e_hi, e_lo], axis=1)

    c16 = counts_ref[...].astype(jnp.bfloat16)
    sb2 = jnp.dot(c16, embp_v[...],
                  preferred_element_type=jnp.float32)
    sb = sb2[:, :2 * _K] + sb2[:, 2 * _K:]
    s = sb[:, :_K]
    ss = sb[:, _K:]
    bi = 0.5 * (s * s - ss)
    h = jnp.maximum(jnp.dot(bi, w1_ref[...]) + b1_ref[...], 0.0)
    h = jnp.maximum(jnp.dot(h, w2_ref[...]) + b2_ref[...], 0.0)
    out_ref[...] = jnp.dot(h, w3_ref[...]) + b3_ref[...]


def _tc_mlp(counts, emb, W1, b1, W2, b2, W3, b3):
    nblk = _B // _BBLK
    full = lambda shape: pl.BlockSpec(shape, lambda i: (0, 0))
    return pl.pallas_call(
        _tc_body,
        grid=(nblk,),
        in_specs=[
            pl.BlockSpec((_BBLK, _FP), lambda i: (i, 0)),
            full((_F, _K)),
            full(W1.shape), full(b1.shape),
            full(W2.shape), full(b2.shape),
            full(W3.shape), full(b3.shape),
        ],
        out_specs=pl.BlockSpec((_BBLK, 1), lambda i: (i, 0)),
        out_shape=jax.ShapeDtypeStruct((_B, 1), jnp.float32),
        scratch_shapes=[pltpu.VMEM((_FP, 4 * _K), jnp.bfloat16)],
    )(counts, emb, W1, b1, W2, b2, W3, b3)


@jax.jit
def kernel(x, emb, W1, b1, W2, b2, W3, b3):
    counts = _sc_counts(x.astype(jnp.int32))
    return _tc_mlp(counts, emb, W1,
                   b1.reshape(1, -1), W2, b2.reshape(1, -1), W3,
                   b3.reshape(1, 1))


# speedup vs baseline: 1.0770x; 1.0027x over previous
import functools

import jax
import jax.numpy as jnp
from jax import lax
from jax.experimental import pallas as pl
from jax.experimental.pallas import tpu as pltpu
from jax.experimental.pallas import tpu_sc as plsc

_B = 1024
_F = 1000
_FP = 1008
_K = 32

_NC = 2
_NS = 16
_NW = _NC * _NS
_ROWS = _B // _NW
_L = 16

_BBLK = 1024


def _sc_hist_body(x_hbm, counts_hbm, x_v, hist_v, sem0, sem1, sem2):
    wid = lax.axis_index("s") * _NC + lax.axis_index("c")
    base = wid * _ROWS
    cp0 = pltpu.async_copy(x_hbm.at[pl.ds(base, _L)], x_v.at[pl.ds(0, _L)],
                           sem0)
    cp1 = pltpu.async_copy(x_hbm.at[pl.ds(base + _L, _L)],
                           x_v.at[pl.ds(_L, _L)], sem1)

    zeros = jnp.zeros((_L,), jnp.float32)

    @plsc.parallel_loop(0, _ROWS, 1, unroll=2)
    def _zero(r):
        for c in range(_FP // _L):
            hist_v[r, pl.ds(c * _L, _L)] = zeros

    ones = jnp.ones((_L,), jnp.float32)
    lane = lax.iota(jnp.int32, _L)

    cp0.wait()

    @plsc.parallel_loop(0, _F, 1, unroll=8)
    def _feat0(f):
        fv = jnp.full((_L,), f, dtype=jnp.int32)
        ids = plsc.load_gather(x_v, [lane, fv])
        plsc.addupdate_scatter(hist_v, [lane, ids], ones)

    out0 = pltpu.async_copy(hist_v.at[pl.ds(0, _L)],
                            counts_hbm.at[pl.ds(base, _L)], sem2)
    cp1.wait()
    rows1 = lane + _L

    @plsc.parallel_loop(0, _F, 1, unroll=8)
    def _feat1(f):
        fv = jnp.full((_L,), f, dtype=jnp.int32)
        ids = plsc.load_gather(x_v, [rows1, fv])
        plsc.addupdate_scatter(hist_v, [rows1, ids], ones)

    out0.wait()
    pltpu.sync_copy(hist_v.at[pl.ds(_L, _L)],
                    counts_hbm.at[pl.ds(base + _L, _L)])


def _sc_counts(x):
    mesh = plsc.VectorSubcoreMesh(core_axis_name="c", subcore_axis_name="s")
    return pl.kernel(
        _sc_hist_body,
        out_type=jax.ShapeDtypeStruct((_B, _FP), jnp.float32),
        mesh=mesh,
        compiler_params=pltpu.CompilerParams(
            use_tc_tiling_on_sc=False, needs_layout_passes=False),
        scratch_types=[
            pltpu.VMEM((_ROWS, _F), jnp.int32),
            pltpu.VMEM((_ROWS, _FP), jnp.float32),
            pltpu.SemaphoreType.DMA,
            pltpu.SemaphoreType.DMA,
            pltpu.SemaphoreType.DMA,
        ],
    )(x)


def _tc_body(counts_ref, emb_ref, w1_ref, b1_ref, w2_ref, b2_ref, w3_ref,
             b3_ref, out_ref, embp_v):
    @pl.when(pl.program_id(0) == 0)
    def _prep():
        emb = emb_ref[...]
        ee = jnp.concatenate([emb, emb * emb], axis=1)
        pad = jnp.zeros((_FP - _F, 2 * _K), jnp.float32)
        eef = jnp.concatenate([ee, pad], axis=0)
        e_hi = eef.astype(jnp.bfloat16)
        e_lo = (eef - e_hi.astype(jnp.float32)).astype(jnp.bfloat16)
        embp_v[...] = jnp.concatenate([e_hi, e_lo], axis=1)

    c16 = counts_ref[...].astype(jnp.bfloat16)
    sb2 = jnp.dot(c16, embp_v[...],
                  preferred_element_type=jnp.float32)
    sb = sb2[:, :2 * _K] + sb2[:, 2 * _K:]
    s = sb[:, :_K]
    ss = sb[:, _K:]
    bi = 0.5 * (s * s - ss)
    h = jnp.maximum(jnp.dot(bi, w1_ref[...]) + b1_ref[...], 0.0)
    h = jnp.maximum(jnp.dot(h, w2_ref[...]) + b2_ref[...], 0.0)
    out_ref[...] = jnp.dot(h, w3_ref[...]) + b3_ref[...]


def _tc_mlp(counts, emb, W1, b1, W2, b2, W3, b3):
    nblk = _B // _BBLK
    full = lambda shape: pl.BlockSpec(shape, lambda i: (0, 0))
    return pl.pallas_call(
        _tc_body,
        grid=(nblk,),
        in_specs=[
            pl.BlockSpec((_BBLK, _FP), lambda i: (i, 0)),
            full((_F, _K)),
            full(W1.shape), full(b1.shape),
            full(W2.shape), full(b2.shape),
            full(W3.shape), full(b3.shape),
        ],
        out_specs=pl.BlockSpec((_BBLK, 1), lambda i: (i, 0)),
        out_shape=jax.ShapeDtypeStruct((_B, 1), jnp.float32),
        scratch_shapes=[pltpu.VMEM((_FP, 4 * _K), jnp.bfloat16)],
    )(counts, emb, W1, b1, W2, b2, W3, b3)


@jax.jit
def kernel(x, emb, W1, b1, W2, b2, W3, b3):
    counts = _sc_counts(x.astype(jnp.int32))
    return _tc_mlp(counts, emb, W1,
                   b1.reshape(1, -1), W2, b2.reshape(1, -1), W3,
                   b3.reshape(1, 1))
